# initial kernel scaffold (unmeasured)
import jax
import jax.numpy as jnp
from jax import lax
from jax.experimental import pallas as pl
from jax.experimental.pallas import tpu as pltpu

T = 2048
D = 4096
V_SHARD = 8192
V = 16384
TILE = 1024
NT = V_SHARD // TILE


def kernel(x, W):
    def body(x_ref, w_ref, out_ref, w_buf, e_buf, sum_ref, s_peer_ref,
             send_sems, recv_sems, copy_sems, w_sems, s_send, s_recv):
        my_x = lax.axis_index("x")
        my_y = lax.axis_index("y")
        my_z = lax.axis_index("z")
        peer = (1 - my_x, my_y, my_z)
        my_off = my_x * V_SHARD
        peer_off = V_SHARD - my_off

        def w_load(i, slot):
            return pltpu.make_async_copy(
                w_ref.at[:, pl.ds(i * TILE, TILE)], w_buf.at[slot],
                w_sems.at[slot])

        w_load(0, 0).start()
        rdmas = []
        sums = jnp.zeros((T, 1), jnp.float32)
        for i in range(NT):
            slot = i % 2
            if i + 1 < NT:
                w_load(i + 1, 1 - slot).start()
            w_load(i, slot).wait()
            logits = lax.dot_general(
                x_ref[...], w_buf[slot],
                (((1,), (0,)), ((), ())),
                preferred_element_type=jnp.float32)
            e = jnp.exp(logits)
            sums = sums + jnp.sum(e, axis=1, keepdims=True)
            e_buf[slot] = e
            out_slice = out_ref.at[:, pl.ds(my_off + i * TILE, TILE)]
            cp = pltpu.make_async_copy(e_buf.at[slot], out_slice,
                                       copy_sems.at[slot])
            cp.start()
            cp.wait()
            rdma = pltpu.make_async_remote_copy(
                src_ref=out_slice,
                dst_ref=out_slice,
                send_sem=send_sems.at[i],
                recv_sem=recv_sems.at[i],
                device_id=peer,
                device_id_type=pl.DeviceIdType.MESH,
            )
            rdma.start()
            rdmas.append(rdma)
        sum_ref[...] = sums

        s_rdma = pltpu.make_async_remote_copy(
            src_ref=sum_ref, dst_ref=s_peer_ref,
            send_sem=s_send, recv_sem=s_recv,
            device_id=peer, device_id_type=pl.DeviceIdType.MESH)
        s_rdma.start()
        s_rdma.wait()
        inv = 1.0 / (sum_ref[...] + s_peer_ref[...])

        for i in range(NT):
            rdmas[i].wait_send()
            mine = out_ref.at[:, pl.ds(my_off + i * TILE, TILE)]
            ld = pltpu.make_async_copy(mine, e_buf.at[0], copy_sems.at[0])
            ld.start()
            ld.wait()
            e_buf[0] = e_buf[0] * inv
            st = pltpu.make_async_copy(e_buf.at[0], mine, copy_sems.at[0])
            st.start()
            st.wait()

            rdmas[i].wait_recv()
            theirs = out_ref.at[:, pl.ds(peer_off + i * TILE, TILE)]
            ld2 = pltpu.make_async_copy(theirs, e_buf.at[1], copy_sems.at[1])
            ld2.start()
            ld2.wait()
            e_buf[1] = e_buf[1] * inv
            st2 = pltpu.make_async_copy(e_buf.at[1], theirs, copy_sems.at[1])
            st2.start()
            st2.wait()

    return pl.pallas_call(
        body,
        out_shape=jax.ShapeDtypeStruct((T, V), jnp.float32),
        in_specs=[
            pl.BlockSpec(memory_space=pltpu.VMEM),
            pl.BlockSpec(memory_space=pltpu.ANY),
        ],
        out_specs=pl.BlockSpec(memory_space=pltpu.ANY),
        scratch_shapes=[
            pltpu.VMEM((2, D, TILE), jnp.float32),
            pltpu.VMEM((2, T, TILE), jnp.float32),
            pltpu.VMEM((T, 1), jnp.float32),
            pltpu.VMEM((T, 1), jnp.float32),
            pltpu.SemaphoreType.DMA((NT,)),
            pltpu.SemaphoreType.DMA((NT,)),
            pltpu.SemaphoreType.DMA((2,)),
            pltpu.SemaphoreType.DMA((2,)),
            pltpu.SemaphoreType.DMA,
            pltpu.SemaphoreType.DMA,
        ],
    )(x, W)


# baseline (device time: 984262 ns/iter reference)
import jax
import jax.numpy as jnp
from jax import lax
from jax.experimental import pallas as pl
from jax.experimental.pallas import tpu as pltpu

T = 2048
D = 4096
V_SHARD = 8192
V = 16384
TILE = 1024
NT = V_SHARD // TILE
WCHUNK = 512
NW = TILE // WCHUNK
NCHUNK = V_SHARD // WCHUNK


def kernel(x, W):
    def body(x_ref, w_ref, out_ref, w_buf, e_buf, sum_ref, s_peer_ref,
             send_sems, recv_sems, copy_sem, w_sem, s_send, s_recv):
        my_x = lax.axis_index("x")
        my_y = lax.axis_index("y")
        my_z = lax.axis_index("z")
        peer = (1 - my_x, my_y, my_z)
        my_off = my_x * V_SHARD
        peer_off = V_SHARD - my_off

        def tile_rdma(i):
            out_slice = out_ref.at[:, pl.ds(my_off + i * TILE, TILE)]
            return pltpu.make_async_remote_copy(
                src_ref=out_slice,
                dst_ref=out_slice,
                send_sem=send_sems.at[i],
                recv_sem=recv_sems.at[i],
                device_id=peer,
                device_id_type=pl.DeviceIdType.MESH,
            )

        sum_ref[...] = jnp.zeros((T, 1), jnp.float32)

        def chunk_step(c, _):
            wl = pltpu.make_async_copy(
                w_ref.at[:, pl.ds(c * WCHUNK, WCHUNK)], w_buf, w_sem)
            wl.start()
            wl.wait()
            logits = lax.dot_general(
                x_ref[...], w_buf[...],
                (((1,), (0,)), ((), ())),
                preferred_element_type=jnp.float32)
            e = jnp.exp(logits)
            sum_ref[...] = sum_ref[...] + jnp.sum(e, axis=1, keepdims=True)
            e_buf[:, pl.ds((c % NW) * WCHUNK, WCHUNK)] = e

            @pl.when(c % NW == NW - 1)
            def _():
                i = c // NW
                out_slice = out_ref.at[:, pl.ds(my_off + i * TILE, TILE)]
                cp = pltpu.make_async_copy(e_buf, out_slice, copy_sem)
                cp.start()
                cp.wait()
                tile_rdma(i).start()

            return _

        lax.fori_loop(0, NCHUNK, chunk_step, None)

        s_rdma = pltpu.make_async_remote_copy(
            src_ref=sum_ref, dst_ref=s_peer_ref,
            send_sem=s_send, recv_sem=s_recv,
            device_id=peer, device_id_type=pl.DeviceIdType.MESH)
        s_rdma.start()
        s_rdma.wait()
        inv = 1.0 / (sum_ref[...] + s_peer_ref[...])

        def norm_step(i, _):
            rdma = tile_rdma(i)
            rdma.wait_send()
            mine = out_ref.at[:, pl.ds(my_off + i * TILE, TILE)]
            ld = pltpu.make_async_copy(mine, e_buf, copy_sem)
            ld.start()
            ld.wait()
            e_buf[...] = e_buf[...] * inv
            st = pltpu.make_async_copy(e_buf, mine, copy_sem)
            st.start()
            st.wait()

            rdma.wait_recv()
            theirs = out_ref.at[:, pl.ds(peer_off + i * TILE, TILE)]
            ld2 = pltpu.make_async_copy(theirs, e_buf, copy_sem)
            ld2.start()
            ld2.wait()
            e_buf[...] = e_buf[...] * inv
            st2 = pltpu.make_async_copy(e_buf, theirs, copy_sem)
            st2.start()
            st2.wait()
            return _

        lax.fori_loop(0, NT, norm_step, None)

    return pl.pallas_call(
        body,
        out_shape=jax.ShapeDtypeStruct((T, V), jnp.float32),
        in_specs=[
            pl.BlockSpec(memory_space=pltpu.VMEM),
            pl.BlockSpec(memory_space=pl.ANY),
        ],
        out_specs=pl.BlockSpec(memory_space=pl.ANY),
        scratch_shapes=[
            pltpu.VMEM((D, WCHUNK), jnp.float32),
            pltpu.VMEM((T, TILE), jnp.float32),
            pltpu.VMEM((T, 1), jnp.float32),
            pltpu.VMEM((T, 1), jnp.float32),
            pltpu.SemaphoreType.DMA((NT,)),
            pltpu.SemaphoreType.DMA((NT,)),
            pltpu.SemaphoreType.DMA,
            pltpu.SemaphoreType.DMA,
            pltpu.SemaphoreType.DMA,
            pltpu.SemaphoreType.DMA,
        ],
        compiler_params=pltpu.CompilerParams(
            vmem_limit_bytes=64 * 1024 * 1024,
        ),
    )(x, W)


# device time: 981556 ns/iter; 1.0028x vs baseline; 1.0028x over previous
import jax
import jax.numpy as jnp
from jax import lax
from jax.experimental import pallas as pl
from jax.experimental.pallas import tpu as pltpu

T = 2048
D = 4096
V_SHARD = 8192
V = 16384
TILE = 1024
NT = V_SHARD // TILE
WCHUNK = 512
NW = TILE // WCHUNK
NCHUNK = V_SHARD // WCHUNK


def kernel(x, W):
    def body(x_ref, w_ref, out_ref, w_buf, e_buf, sum_ref, s_peer_ref,
             send_sems, recv_sems, copy_sem, w_sems, s_send, s_recv):
        my_x = lax.axis_index("x")
        my_y = lax.axis_index("y")
        my_z = lax.axis_index("z")
        peer = (1 - my_x, my_y, my_z)
        my_off = my_x * V_SHARD
        peer_off = V_SHARD - my_off

        def tile_rdma(i):
            out_slice = out_ref.at[:, pl.ds(my_off + i * TILE, TILE)]
            return pltpu.make_async_remote_copy(
                src_ref=out_slice,
                dst_ref=out_slice,
                send_sem=send_sems.at[i],
                recv_sem=recv_sems.at[i],
                device_id=peer,
                device_id_type=pl.DeviceIdType.MESH,
            )

        sum_ref[...] = jnp.zeros((T, 1), jnp.float32)

        def w_load(c, slot):
            return pltpu.make_async_copy(
                w_ref.at[:, pl.ds(c * WCHUNK, WCHUNK)], w_buf.at[slot],
                w_sems.at[slot])

        with jax.named_scope("phaseA_gemm_send"):
            w_load(0, 0).start()

            def chunk_step(c, _):
                slot = lax.rem(c, 2)

                @pl.when(c + 1 < NCHUNK)
                def _():
                    w_load(c + 1, 1 - slot).start()

                w_load(c, slot).wait()
                logits = lax.dot_general(
                    x_ref[...], w_buf[slot],
                    (((1,), (0,)), ((), ())),
                    preferred_element_type=jnp.float32)
                e = jnp.exp(logits)
                sum_ref[...] = sum_ref[...] + jnp.sum(e, axis=1,
                                                      keepdims=True)
                e_buf[:, pl.ds((c % NW) * WCHUNK, WCHUNK)] = e

                @pl.when(c % NW == NW - 1)
                def _():
                    i = c // NW
                    out_slice = out_ref.at[:, pl.ds(my_off + i * TILE, TILE)]
                    cp = pltpu.make_async_copy(e_buf, out_slice, copy_sem)
                    cp.start()
                    cp.wait()
                    tile_rdma(i).start()

                return _

            lax.fori_loop(0, NCHUNK, chunk_step, None)

        with jax.named_scope("phaseB_sums"):
            s_rdma = pltpu.make_async_remote_copy(
                src_ref=sum_ref, dst_ref=s_peer_ref,
                send_sem=s_send, recv_sem=s_recv,
                device_id=peer, device_id_type=pl.DeviceIdType.MESH)
            s_rdma.start()
            s_rdma.wait()
        inv = 1.0 / (sum_ref[...] + s_peer_ref[...])

        def norm_step(i, _):
            rdma = tile_rdma(i)
            rdma.wait_send()
            mine = out_ref.at[:, pl.ds(my_off + i * TILE, TILE)]
            ld = pltpu.make_async_copy(mine, e_buf, copy_sem)
            ld.start()
            ld.wait()
            e_buf[...] = e_buf[...] * inv
            st = pltpu.make_async_copy(e_buf, mine, copy_sem)
            st.start()
            st.wait()

            rdma.wait_recv()
            theirs = out_ref.at[:, pl.ds(peer_off + i * TILE, TILE)]
            ld2 = pltpu.make_async_copy(theirs, e_buf, copy_sem)
            ld2.start()
            ld2.wait()
            e_buf[...] = e_buf[...] * inv
            st2 = pltpu.make_async_copy(e_buf, theirs, copy_sem)
            st2.start()
            st2.wait()
            return _

        with jax.named_scope("phaseC_normalize"):
            lax.fori_loop(0, NT, norm_step, None)

    return pl.pallas_call(
        body,
        out_shape=jax.ShapeDtypeStruct((T, V), jnp.float32),
        in_specs=[
            pl.BlockSpec(memory_space=pltpu.VMEM),
            pl.BlockSpec(memory_space=pl.ANY),
        ],
        out_specs=pl.BlockSpec(memory_space=pl.ANY),
        scratch_shapes=[
            pltpu.VMEM((2, D, WCHUNK), jnp.float32),
            pltpu.VMEM((T, TILE), jnp.float32),
            pltpu.VMEM((T, 1), jnp.float32),
            pltpu.VMEM((T, 1), jnp.float32),
            pltpu.SemaphoreType.DMA((NT,)),
            pltpu.SemaphoreType.DMA((NT,)),
            pltpu.SemaphoreType.DMA,
            pltpu.SemaphoreType.DMA((2,)),
            pltpu.SemaphoreType.DMA,
            pltpu.SemaphoreType.DMA,
        ],
        compiler_params=pltpu.CompilerParams(
            vmem_limit_bytes=64 * 1024 * 1024,
        ),
    )(x, W)


# device time: 975151 ns/iter; 1.0093x vs baseline; 1.0066x over previous
import jax
import jax.numpy as jnp
from jax import lax
from jax.experimental import pallas as pl
from jax.experimental.pallas import tpu as pltpu

T = 2048
D = 4096
V_SHARD = 8192
V = 16384
TILE = 1024
NT = V_SHARD // TILE
WCHUNK = 512
NW = TILE // WCHUNK
NCHUNK = V_SHARD // WCHUNK


def kernel(x, W):
    def body(x_ref, w_ref, out_ref, comm_ref, w_buf, e_buf, sum_ref,
             s_peer_ref, send_sems, recv_sems, copy_sem, w_sems,
             s_send, s_recv):
        my_x = lax.axis_index("x")
        my_y = lax.axis_index("y")
        my_z = lax.axis_index("z")
        peer = (1 - my_x, my_y, my_z)
        my_off = my_x * V_SHARD
        peer_off = V_SHARD - my_off

        def tile_rdma(i):
            return pltpu.make_async_remote_copy(
                src_ref=comm_ref.at[0, i],
                dst_ref=comm_ref.at[1, i],
                send_sem=send_sems.at[i],
                recv_sem=recv_sems.at[i],
                device_id=peer,
                device_id_type=pl.DeviceIdType.MESH,
            )

        sum_ref[...] = jnp.zeros((T, 1), jnp.float32)

        def w_load(c, slot):
            return pltpu.make_async_copy(
                w_ref.at[:, pl.ds(c * WCHUNK, WCHUNK)], w_buf.at[slot],
                w_sems.at[slot])

        w_load(0, 0).start()

        def chunk_step(c, _):
            slot = lax.rem(c, 2)

            @pl.when(c + 1 < NCHUNK)
            def _():
                w_load(c + 1, 1 - slot).start()

            w_load(c, slot).wait()
            logits = lax.dot_general(
                x_ref[...], w_buf[slot],
                (((1,), (0,)), ((), ())),
                preferred_element_type=jnp.float32)
            e = jnp.exp(logits)
            sum_ref[...] = sum_ref[...] + jnp.sum(e, axis=1, keepdims=True)
            e_buf[:, pl.ds((c % NW) * WCHUNK, WCHUNK)] = e

            @pl.when(c % NW == NW - 1)
            def _():
                i = c // NW
                cp = pltpu.make_async_copy(e_buf, comm_ref.at[0, i],
                                           copy_sem)
                cp.start()
                cp.wait()
                tile_rdma(i).start()

            return _

        lax.fori_loop(0, NCHUNK, chunk_step, None)

        s_rdma = pltpu.make_async_remote_copy(
            src_ref=sum_ref, dst_ref=s_peer_ref,
            send_sem=s_send, recv_sem=s_recv,
            device_id=peer, device_id_type=pl.DeviceIdType.MESH)
        s_rdma.start()
        s_rdma.wait()
        inv = 1.0 / (sum_ref[...] + s_peer_ref[...])

        def norm_step(i, _):
            ld = pltpu.make_async_copy(comm_ref.at[0, i], e_buf, copy_sem)
            ld.start()
            ld.wait()
            e_buf[...] = e_buf[...] * inv
            mine = out_ref.at[:, pl.ds(my_off + i * TILE, TILE)]
            st = pltpu.make_async_copy(e_buf, mine, copy_sem)
            st.start()
            st.wait()

            tile_rdma(i).wait_recv()
            ld2 = pltpu.make_async_copy(comm_ref.at[1, i], e_buf, copy_sem)
            ld2.start()
            ld2.wait()
            e_buf[...] = e_buf[...] * inv
            theirs = out_ref.at[:, pl.ds(peer_off + i * TILE, TILE)]
            st2 = pltpu.make_async_copy(e_buf, theirs, copy_sem)
            st2.start()
            st2.wait()
            return _

        lax.fori_loop(0, NT, norm_step, None)

        def drain(i, _):
            tile_rdma(i).wait_send()
            return _

        lax.fori_loop(0, NT, drain, None)

    return pl.pallas_call(
        body,
        out_shape=[
            jax.ShapeDtypeStruct((T, V), jnp.float32),
            jax.ShapeDtypeStruct((2, NT, T, TILE), jnp.float32),
        ],
        in_specs=[
            pl.BlockSpec(memory_space=pltpu.VMEM),
            pl.BlockSpec(memory_space=pl.ANY),
        ],
        out_specs=[
            pl.BlockSpec(memory_space=pl.ANY),
            pl.BlockSpec(memory_space=pl.ANY),
        ],
        scratch_shapes=[
            pltpu.VMEM((2, D, WCHUNK), jnp.float32),
            pltpu.VMEM((T, TILE), jnp.float32),
            pltpu.VMEM((T, 1), jnp.float32),
            pltpu.VMEM((T, 1), jnp.float32),
            pltpu.SemaphoreType.DMA((NT,)),
            pltpu.SemaphoreType.DMA((NT,)),
            pltpu.SemaphoreType.DMA,
            pltpu.SemaphoreType.DMA((2,)),
            pltpu.SemaphoreType.DMA,
            pltpu.SemaphoreType.DMA,
        ],
        compiler_params=pltpu.CompilerParams(
            vmem_limit_bytes=64 * 1024 * 1024,
        ),
    )(x, W)[0]
